# baseline (speedup 1.0000x reference)
import functools

import jax
import jax.numpy as jnp
from jax import lax
from jax.experimental import pallas as pl
from jax.experimental.pallas import tpu as pltpu
from jax.experimental.pallas import tpu_sc as plsc

N = 10000
E = 320000
D = 128

NC = 2
NS = 16
EPT = E // (NC * NS)
K = 40
NCHUNK = EPT // K
N_PAD = 10240
RPT = N_PAD // NS
ZR = 160
DEG_W = 128

_mesh = plsc.VectorSubcoreMesh(core_axis_name="c", subcore_axis_name="s")


def _zero_fill(buf, rows, width):
    zv = jnp.zeros((16,), jnp.float32)

    def body(i, _):
        for j in range(width // 16):
            buf[i, pl.ds(j * 16, 16)] = zv
        return 0

    lax.fori_loop(0, rows, body, 0)


G = 4
NSUPER = 31
TAIL = NCHUNK - 2 * G * NSUPER
DEG_G = 5


@functools.partial(
    pl.kernel,
    out_type=jax.ShapeDtypeStruct((NC, N_PAD, DEG_W), jnp.float32),
    mesh=_mesh,
    scratch_types=[
        pltpu.VMEM_SHARED((N_PAD, DEG_W), jnp.float32),
        pltpu.VMEM((NCHUNK, K), jnp.int32),
        pltpu.VMEM((K, DEG_W), jnp.float32),
        pltpu.VMEM((K, DEG_W), jnp.float32),
        pltpu.SemaphoreType.DMA,
        pltpu.SemaphoreType.DMA,
    ],
)
def _deg_kernel(dstr_hbm, out_hbm, acc_sh, idx_all, ones_v, zbuf, sa, sb):
    cid = lax.axis_index("c")
    sid = lax.axis_index("s")
    wid = cid * NS + sid
    ov = jnp.ones((16,), jnp.float32)

    def fill_ones(i, _):
        for j in range(DEG_W // 16):
            ones_v[i, pl.ds(j * 16, 16)] = ov
        return 0

    lax.fori_loop(0, K, fill_ones, 0)
    _zero_fill(zbuf, K, DEG_W)
    pltpu.async_copy(dstr_hbm.at[wid], idx_all, sa).wait()

    base_r = sid * RPT
    for t in range(RPT // K):
        pltpu.sync_copy(zbuf, acc_sh.at[pl.ds(base_r + t * K, K)])
    plsc.subcore_barrier()

    def wave(w, _):
        descs = [pltpu.async_copy(
            ones_v, acc_sh.at[idx_all.at[w * DEG_G + c]], sa, add=True)
            for c in range(DEG_G)]
        for d in descs:
            d.wait()
        return 0

    lax.fori_loop(0, NCHUNK // DEG_G, wave, 0)
    plsc.subcore_barrier()
    pltpu.sync_copy(acc_sh.at[pl.ds(base_r, RPT)],
                    out_hbm.at[cid, pl.ds(base_r, RPT)])


@functools.partial(
    pl.kernel,
    out_type=jax.ShapeDtypeStruct((NC, N_PAD, D), jnp.float32),
    mesh=_mesh,
    scratch_types=[
        pltpu.VMEM_SHARED((N_PAD, D), jnp.float32),
        pltpu.VMEM((2 * G, K), jnp.int32),
        pltpu.VMEM((2 * G, K), jnp.int32),
        pltpu.VMEM((2 * G, K), jnp.int32),
        pltpu.VMEM((2 * G, K), jnp.int32),
    ] + [pltpu.VMEM((K, D), jnp.float32) for _ in range(2 * G)] + [
        pltpu.SemaphoreType.DMA,
        pltpu.SemaphoreType.DMA,
        pltpu.SemaphoreType.DMA,
        pltpu.SemaphoreType.DMA,
        pltpu.SemaphoreType.DMA,
    ],
)
def _edge_kernel(srcr_hbm, dstr_hbm, g_hbm, out_hbm,
                 acc_sh, p_s, p_d, q_s, q_d,
                 ra0, ra1, ra2, ra3, rb0, rb1, rb2, rb3,
                 isem, gsa, gsb, ssa, ssb):
    cid = lax.axis_index("c")
    sid = lax.axis_index("s")
    wid = cid * NS + sid
    rows_a = [ra0, ra1, ra2, ra3]
    rows_b = [rb0, rb1, rb2, rb3]

    _zero_fill(ra0, K, D)
    base_r = sid * RPT
    for t in range(RPT // K):
        pltpu.sync_copy(ra0, acc_sh.at[pl.ds(base_r + t * K, K)])
    plsc.subcore_barrier()

    SS = 2 * G
    rows = rows_a + rows_b

    def fire_idx(c0, idxs, idxd):
        pltpu.async_copy(srcr_hbm.at[wid, pl.ds(c0, SS)], idxs, isem)
        pltpu.async_copy(dstr_hbm.at[wid, pl.ds(c0, SS)], idxd, isem)

    def wait_idx(c0, idxs, idxd):
        pltpu.make_async_copy(srcr_hbm.at[wid, pl.ds(c0, SS)], idxs,
                              isem).wait()
        pltpu.make_async_copy(dstr_hbm.at[wid, pl.ds(c0, SS)], idxd,
                              isem).wait()

    def drain_sc(idxd):
        for c in range(SS):
            pltpu.make_async_copy(rows[c], acc_sh.at[idxd.at[c]], ssa).wait()

    def superstep(c0, idxs, idxd, prevd, drains, c0n, nidxs, nidxd):
        if drains:
            drain_sc(prevd)
        wait_idx(c0, idxs, idxd)
        if nidxs is not None:
            fire_idx(c0n, nidxs, nidxd)
        ga = [pltpu.async_copy(g_hbm.at[idxs.at[c]], rows[c], gsa)
              for c in range(SS)]
        for dsc in ga:
            dsc.wait()
        for c in range(SS):
            pltpu.async_copy(rows[c], acc_sh.at[idxd.at[c]], ssa, add=True)

    fire_idx(0, p_s, p_d)
    superstep(0, p_s, p_d, None, False, SS, q_s, q_d)

    def dbl(i, _):
        cq = pl.multiple_of(2 * SS * i + SS, 8)
        superstep(cq, q_s, q_d, p_d, True, cq + SS, p_s, p_d)
        superstep(cq + SS, p_s, p_d, q_d, True, cq + 2 * SS, q_s, q_d)
        return 0

    lax.fori_loop(0, (NSUPER - 3) // 2, dbl, 0)

    c29 = pl.multiple_of((NSUPER - 2) * SS, 8)
    superstep(c29, q_s, q_d, p_d, True, c29 + SS, p_s, p_d)
    superstep(c29 + SS, p_s, p_d, q_d, True, 0, None, None)
    drain_sc(p_d)

    dt1 = pltpu.async_copy(
        srcr_hbm.at[wid, pl.ds(pl.multiple_of(NSUPER * SS, 8), TAIL)],
        q_s.at[pl.ds(0, TAIL)], isem)
    dt2 = pltpu.async_copy(
        dstr_hbm.at[wid, pl.ds(pl.multiple_of(NSUPER * SS, 8), TAIL)],
        q_d.at[pl.ds(0, TAIL)], isem)
    dt1.wait()
    dt2.wait()
    gt = [pltpu.async_copy(g_hbm.at[q_s.at[c]], rows_a[c], gsa)
          for c in range(TAIL)]
    for dsc in gt:
        dsc.wait()
    st = [pltpu.async_copy(rows_a[c], acc_sh.at[q_d.at[c]], ssa, add=True)
          for c in range(TAIL)]
    for dsc in st:
        dsc.wait()

    plsc.subcore_barrier()
    pltpu.sync_copy(acc_sh.at[pl.ds(base_r, RPT)],
                    out_hbm.at[cid, pl.ds(base_r, RPT)])


R_TC = 2000


def _dinv_block(dp_ref):
    deg = dp_ref[0, :, 0:1] + dp_ref[1, :, 0:1] + 1.0
    return lax.rsqrt(deg)


def _tc1_body(x_ref, w_ref, dp_ref, o_ref):
    dinv = _dinv_block(dp_ref)
    h = jnp.dot(x_ref[...], w_ref[...], preferred_element_type=jnp.float32)
    o_ref[...] = h * dinv


def _tc2_body(acc_ref, g_ref, dp_ref, w_ref, b_ref, o_ref):
    dinv = _dinv_block(dp_ref)
    s = acc_ref[0] + acc_ref[1] + g_ref[...]
    h = jnp.maximum(s * dinv + b_ref[...], 0.0)
    o_ref[...] = jnp.dot(h, w_ref[...], preferred_element_type=jnp.float32) * dinv


def _tc3_body(acc_ref, g_ref, dp_ref, b_ref, o_ref):
    dinv = _dinv_block(dp_ref)
    s = acc_ref[0] + acc_ref[1] + g_ref[...]
    o_ref[...] = s * dinv + b_ref[...]


_row_spec = pl.BlockSpec((R_TC, D), lambda i: (i, 0))
_acc_spec = pl.BlockSpec((NC, R_TC, D), lambda i: (0, i, 0))
_dp_spec = pl.BlockSpec((NC, R_TC, DEG_W), lambda i: (0, i, 0))
_w_spec = pl.BlockSpec((D, D), lambda i: (0, 0))
_b_spec = pl.BlockSpec((1, D), lambda i: (0, 0))
_grid = (N // R_TC,)
_out_t = jax.ShapeDtypeStruct((N, D), jnp.float32)

_tc1 = pl.pallas_call(
    _tc1_body, grid=_grid,
    in_specs=[_row_spec, _w_spec, _dp_spec],
    out_specs=_row_spec, out_shape=_out_t)

_tc2 = pl.pallas_call(
    _tc2_body, grid=_grid,
    in_specs=[_acc_spec, _row_spec, _dp_spec, _w_spec, _b_spec],
    out_specs=_row_spec, out_shape=_out_t)

_tc3 = pl.pallas_call(
    _tc3_body, grid=_grid,
    in_specs=[_acc_spec, _row_spec, _dp_spec, _b_spec],
    out_specs=_row_spec, out_shape=_out_t)


def kernel(x, edge_index, W1, b1, W2, b2):
    nw = NC * NS
    src = edge_index[0].reshape(nw, NCHUNK, K)
    dst = edge_index[1].reshape(nw, NCHUNK, K)
    dp = _deg_kernel(dst)
    g1 = _tc1(x, W1, dp)
    acc1 = _edge_kernel(src, dst, g1)
    g2 = _tc2(acc1, g1, dp, W2, b1.reshape(1, D))
    acc2 = _edge_kernel(src, dst, g2)
    return _tc3(acc2, g2, dp, b2.reshape(1, D))


# speedup vs baseline: 1.0241x; 1.0241x over previous
import functools

import jax
import jax.numpy as jnp
from jax import lax
from jax.experimental import pallas as pl
from jax.experimental.pallas import tpu as pltpu
from jax.experimental.pallas import tpu_sc as plsc

N = 10000
E = 320000
D = 128

NC = 2
NS = 16
EPT = E // (NC * NS)
K = 40
NCHUNK = EPT // K
N_PAD = 10240
RPT = N_PAD // NS
ZR = 160
DEG_W = 128

_mesh = plsc.VectorSubcoreMesh(core_axis_name="c", subcore_axis_name="s")


def _zero_fill(buf, rows, width):
    zv = jnp.zeros((16,), jnp.float32)

    def body(i, _):
        for j in range(width // 16):
            buf[i, pl.ds(j * 16, 16)] = zv
        return 0

    lax.fori_loop(0, rows, body, 0)


G = 4
NSUPER = 31
TAIL = NCHUNK - 2 * G * NSUPER
DEG_G = 5


@functools.partial(
    pl.kernel,
    out_type=jax.ShapeDtypeStruct((NC, N_PAD, DEG_W), jnp.float32),
    mesh=_mesh,
    scratch_types=[
        pltpu.VMEM_SHARED((N_PAD, DEG_W), jnp.float32),
        pltpu.VMEM((NCHUNK, K), jnp.int32),
        pltpu.VMEM((K, DEG_W), jnp.float32),
        pltpu.VMEM((K, DEG_W), jnp.float32),
        pltpu.SemaphoreType.DMA,
        pltpu.SemaphoreType.DMA,
    ],
)
def _deg_kernel(dstr_hbm, out_hbm, acc_sh, idx_all, ones_v, zbuf, sa, sb):
    cid = lax.axis_index("c")
    sid = lax.axis_index("s")
    wid = cid * NS + sid
    ov = jnp.ones((16,), jnp.float32)

    def fill_ones(i, _):
        for j in range(DEG_W // 16):
            ones_v[i, pl.ds(j * 16, 16)] = ov
        return 0

    lax.fori_loop(0, K, fill_ones, 0)
    _zero_fill(zbuf, K, DEG_W)
    pltpu.async_copy(dstr_hbm.at[wid], idx_all, sa).wait()

    base_r = sid * RPT
    for t in range(RPT // K):
        pltpu.sync_copy(zbuf, acc_sh.at[pl.ds(base_r + t * K, K)])
    plsc.subcore_barrier()

    def wave(w, _):
        descs = [pltpu.async_copy(
            ones_v, acc_sh.at[idx_all.at[w * DEG_G + c]], sa, add=True)
            for c in range(DEG_G)]
        for d in descs:
            d.wait()
        return 0

    lax.fori_loop(0, NCHUNK // DEG_G, wave, 0)
    plsc.subcore_barrier()
    pltpu.sync_copy(acc_sh.at[pl.ds(base_r, RPT)],
                    out_hbm.at[cid, pl.ds(base_r, RPT)])


@functools.partial(
    pl.kernel,
    out_type=jax.ShapeDtypeStruct((NC, N_PAD, D), jnp.float32),
    mesh=_mesh,
    scratch_types=[
        pltpu.VMEM_SHARED((N_PAD, D), jnp.float32),
        pltpu.VMEM((2 * G, K), jnp.int32),
        pltpu.VMEM((2 * G, K), jnp.int32),
        pltpu.VMEM((2 * G, K), jnp.int32),
        pltpu.VMEM((2 * G, K), jnp.int32),
    ] + [pltpu.VMEM((K, D), jnp.float32) for _ in range(2 * G)] + [
        pltpu.SemaphoreType.DMA,
        pltpu.SemaphoreType.DMA,
        pltpu.SemaphoreType.DMA,
        pltpu.SemaphoreType.DMA,
        pltpu.SemaphoreType.DMA,
    ],
)
def _edge_kernel(srcr_hbm, dstr_hbm, g_hbm, out_hbm,
                 acc_sh, p_s, p_d, q_s, q_d,
                 ra0, ra1, ra2, ra3, rb0, rb1, rb2, rb3,
                 isem, gsa, gsb, ssa, ssb):
    cid = lax.axis_index("c")
    sid = lax.axis_index("s")
    wid = cid * NS + sid
    rows_a = [ra0, ra1, ra2, ra3]
    rows_b = [rb0, rb1, rb2, rb3]

    _zero_fill(ra0, K, D)
    base_r = sid * RPT
    for t in range(RPT // K):
        pltpu.sync_copy(ra0, acc_sh.at[pl.ds(base_r + t * K, K)])
    plsc.subcore_barrier()

    SS = 2 * G
    rows = rows_a + rows_b

    def fire_idx(c0, idxs, idxd):
        pltpu.async_copy(srcr_hbm.at[wid, pl.ds(c0, SS)], idxs, isem)
        pltpu.async_copy(dstr_hbm.at[wid, pl.ds(c0, SS)], idxd, isem)

    def wait_idx(c0, idxs, idxd):
        pltpu.make_async_copy(srcr_hbm.at[wid, pl.ds(c0, SS)], idxs,
                              isem).wait()
        pltpu.make_async_copy(dstr_hbm.at[wid, pl.ds(c0, SS)], idxd,
                              isem).wait()

    def drain_a(idxd):
        for c in range(G):
            pltpu.make_async_copy(rows[c], acc_sh.at[idxd.at[c]], ssa).wait()

    def drain_b(idxd):
        for c in range(G):
            pltpu.make_async_copy(rows[G + c], acc_sh.at[idxd.at[G + c]],
                                  ssb).wait()

    def superstep(c0, idxs, idxd, prevd, drains, c0n, nidxs, nidxd):
        if drains:
            drain_a(prevd)
        wait_idx(c0, idxs, idxd)
        ga = [pltpu.async_copy(g_hbm.at[idxs.at[c]], rows[c], gsa)
              for c in range(G)]
        if drains:
            drain_b(prevd)
        gb = [pltpu.async_copy(g_hbm.at[idxs.at[G + c]], rows[G + c], gsb)
              for c in range(G)]
        if nidxs is not None:
            fire_idx(c0n, nidxs, nidxd)
        for dsc in ga:
            dsc.wait()
        for c in range(G):
            pltpu.async_copy(rows[c], acc_sh.at[idxd.at[c]], ssa, add=True)
        for dsc in gb:
            dsc.wait()
        for c in range(G):
            pltpu.async_copy(rows[G + c], acc_sh.at[idxd.at[G + c]], ssb,
                             add=True)

    fire_idx(0, p_s, p_d)
    superstep(0, p_s, p_d, None, False, SS, q_s, q_d)

    def dbl(i, _):
        cq = pl.multiple_of(2 * SS * i + SS, 8)
        superstep(cq, q_s, q_d, p_d, True, cq + SS, p_s, p_d)
        superstep(cq + SS, p_s, p_d, q_d, True, cq + 2 * SS, q_s, q_d)
        return 0

    lax.fori_loop(0, (NSUPER - 3) // 2, dbl, 0)

    c29 = pl.multiple_of((NSUPER - 2) * SS, 8)
    superstep(c29, q_s, q_d, p_d, True, c29 + SS, p_s, p_d)
    superstep(c29 + SS, p_s, p_d, q_d, True, 0, None, None)
    drain_a(p_d)
    drain_b(p_d)

    dt1 = pltpu.async_copy(
        srcr_hbm.at[wid, pl.ds(pl.multiple_of(NSUPER * SS, 8), TAIL)],
        q_s.at[pl.ds(0, TAIL)], isem)
    dt2 = pltpu.async_copy(
        dstr_hbm.at[wid, pl.ds(pl.multiple_of(NSUPER * SS, 8), TAIL)],
        q_d.at[pl.ds(0, TAIL)], isem)
    dt1.wait()
    dt2.wait()
    gt = [pltpu.async_copy(g_hbm.at[q_s.at[c]], rows_a[c], gsa)
          for c in range(TAIL)]
    for dsc in gt:
        dsc.wait()
    st = [pltpu.async_copy(rows_a[c], acc_sh.at[q_d.at[c]], ssa, add=True)
          for c in range(TAIL)]
    for dsc in st:
        dsc.wait()

    plsc.subcore_barrier()
    pltpu.sync_copy(acc_sh.at[pl.ds(base_r, RPT)],
                    out_hbm.at[cid, pl.ds(base_r, RPT)])


R_TC = 2000


def _dinv_block(dp_ref):
    deg = dp_ref[0, :, 0:1] + dp_ref[1, :, 0:1] + 1.0
    return lax.rsqrt(deg)


def _tc1_body(x_ref, w_ref, dp_ref, o_ref):
    dinv = _dinv_block(dp_ref)
    h = jnp.dot(x_ref[...], w_ref[...], preferred_element_type=jnp.float32)
    o_ref[...] = h * dinv


def _tc2_body(acc_ref, g_ref, dp_ref, w_ref, b_ref, o_ref):
    dinv = _dinv_block(dp_ref)
    s = acc_ref[0] + acc_ref[1] + g_ref[...]
    h = jnp.maximum(s * dinv + b_ref[...], 0.0)
    o_ref[...] = jnp.dot(h, w_ref[...], preferred_element_type=jnp.float32) * dinv


def _tc3_body(acc_ref, g_ref, dp_ref, b_ref, o_ref):
    dinv = _dinv_block(dp_ref)
    s = acc_ref[0] + acc_ref[1] + g_ref[...]
    o_ref[...] = s * dinv + b_ref[...]


_row_spec = pl.BlockSpec((R_TC, D), lambda i: (i, 0))
_acc_spec = pl.BlockSpec((NC, R_TC, D), lambda i: (0, i, 0))
_dp_spec = pl.BlockSpec((NC, R_TC, DEG_W), lambda i: (0, i, 0))
_w_spec = pl.BlockSpec((D, D), lambda i: (0, 0))
_b_spec = pl.BlockSpec((1, D), lambda i: (0, 0))
_grid = (N // R_TC,)
_out_t = jax.ShapeDtypeStruct((N, D), jnp.float32)

_tc1 = pl.pallas_call(
    _tc1_body, grid=_grid,
    in_specs=[_row_spec, _w_spec, _dp_spec],
    out_specs=_row_spec, out_shape=_out_t)

_tc2 = pl.pallas_call(
    _tc2_body, grid=_grid,
    in_specs=[_acc_spec, _row_spec, _dp_spec, _w_spec, _b_spec],
    out_specs=_row_spec, out_shape=_out_t)

_tc3 = pl.pallas_call(
    _tc3_body, grid=_grid,
    in_specs=[_acc_spec, _row_spec, _dp_spec, _b_spec],
    out_specs=_row_spec, out_shape=_out_t)


def kernel(x, edge_index, W1, b1, W2, b2):
    nw = NC * NS
    src = edge_index[0].reshape(nw, NCHUNK, K)
    dst = edge_index[1].reshape(nw, NCHUNK, K)
    dp = _deg_kernel(dst)
    g1 = _tc1(x, W1, dp)
    acc1 = _edge_kernel(src, dst, g1)
    g2 = _tc2(acc1, g1, dp, W2, b1.reshape(1, D))
    acc2 = _edge_kernel(src, dst, g2)
    return _tc3(acc2, g2, dp, b2.reshape(1, D))


# speedup vs baseline: 1.1447x; 1.1178x over previous
import functools

import jax
import jax.numpy as jnp
from jax import lax
from jax.experimental import pallas as pl
from jax.experimental.pallas import tpu as pltpu
from jax.experimental.pallas import tpu_sc as plsc

N = 10000
E = 320000
D = 128

NC = 2
NS = 16
EPT = E // (NC * NS)
K = 40
NCHUNK = EPT // K
N_PAD = 10240
RPT = N_PAD // NS
ZR = 160
DEG_W = 128

_mesh = plsc.VectorSubcoreMesh(core_axis_name="c", subcore_axis_name="s")


def _zero_fill(buf, rows, width):
    zv = jnp.zeros((16,), jnp.float32)

    def body(i, _):
        for j in range(width // 16):
            buf[i, pl.ds(j * 16, 16)] = zv
        return 0

    lax.fori_loop(0, rows, body, 0)


G = 4
NSUPER = 31
TAIL = NCHUNK - 2 * G * NSUPER
DEG_G = 5


NW = NC * NS
HROWS = EPT // 16
HPAD = N_PAD // 128


@functools.partial(
    pl.kernel,
    out_type=jax.ShapeDtypeStruct((NW, HPAD, 128), jnp.float32),
    mesh=_mesh,
    compiler_params=pltpu.CompilerParams(needs_layout_passes=False),
    scratch_types=[
        pltpu.VMEM((HPAD, 128), jnp.float32),
        pltpu.VMEM((HROWS, 16), jnp.int32),
        pltpu.SemaphoreType.DMA,
    ],
)
def _deg_kernel(dstr_hbm, out_hbm, hist, idxv, sem):
    cid = lax.axis_index("c")
    sid = lax.axis_index("s")
    wid = cid * NS + sid
    ld = pltpu.async_copy(dstr_hbm.at[wid], idxv, sem)

    def zrow(i, _):
        for j in range(8):
            hist[i, pl.ds(j * 16, 16)] = jnp.zeros((16,), jnp.float32)
        return 0

    lax.fori_loop(0, HPAD, zrow, 0)
    ld.wait()
    ones = jnp.ones((16,), jnp.float32)

    def row(r, _):
        n = idxv[r, :]
        hi = lax.shift_right_logical(n, 7)
        lo = lax.bitwise_and(n, 127)
        plsc.addupdate_scatter(hist, (hi, lo), ones)
        return 0

    lax.fori_loop(0, HROWS, row, 0)
    pltpu.sync_copy(hist, out_hbm.at[wid])


@functools.partial(
    pl.kernel,
    out_type=jax.ShapeDtypeStruct((NC, N_PAD, D), jnp.float32),
    mesh=_mesh,
    scratch_types=[
        pltpu.VMEM_SHARED((N_PAD, D), jnp.float32),
        pltpu.VMEM((2 * G, K), jnp.int32),
        pltpu.VMEM((2 * G, K), jnp.int32),
        pltpu.VMEM((2 * G, K), jnp.int32),
        pltpu.VMEM((2 * G, K), jnp.int32),
    ] + [pltpu.VMEM((K, D), jnp.float32) for _ in range(2 * G)] + [
        pltpu.SemaphoreType.DMA,
        pltpu.SemaphoreType.DMA,
        pltpu.SemaphoreType.DMA,
        pltpu.SemaphoreType.DMA,
        pltpu.SemaphoreType.DMA,
    ],
)
def _edge_kernel(srcr_hbm, dstr_hbm, g_hbm, out_hbm,
                 acc_sh, p_s, p_d, q_s, q_d,
                 ra0, ra1, ra2, ra3, rb0, rb1, rb2, rb3,
                 isem, gsa, gsb, ssa, ssb):
    cid = lax.axis_index("c")
    sid = lax.axis_index("s")
    wid = cid * NS + sid
    rows_a = [ra0, ra1, ra2, ra3]
    rows_b = [rb0, rb1, rb2, rb3]

    _zero_fill(ra0, K, D)
    base_r = sid * RPT
    for t in range(RPT // K):
        pltpu.sync_copy(ra0, acc_sh.at[pl.ds(base_r + t * K, K)])
    plsc.subcore_barrier()

    SS = 2 * G
    rows = rows_a + rows_b

    def fire_idx(c0, idxs, idxd):
        pltpu.async_copy(srcr_hbm.at[wid, pl.ds(c0, SS)], idxs, isem)
        pltpu.async_copy(dstr_hbm.at[wid, pl.ds(c0, SS)], idxd, isem)

    def wait_idx(c0, idxs, idxd):
        pltpu.make_async_copy(srcr_hbm.at[wid, pl.ds(c0, SS)], idxs,
                              isem).wait()
        pltpu.make_async_copy(dstr_hbm.at[wid, pl.ds(c0, SS)], idxd,
                              isem).wait()

    def drain_a(idxd):
        for c in range(G):
            pltpu.make_async_copy(rows[c], acc_sh.at[idxd.at[c]], ssa).wait()

    def drain_b(idxd):
        for c in range(G):
            pltpu.make_async_copy(rows[G + c], acc_sh.at[idxd.at[G + c]],
                                  ssb).wait()

    def superstep(c0, idxs, idxd, prevd, drains, c0n, nidxs, nidxd):
        if drains:
            drain_a(prevd)
        wait_idx(c0, idxs, idxd)
        ga = [pltpu.async_copy(g_hbm.at[idxs.at[c]], rows[c], gsa)
              for c in range(G)]
        if drains:
            drain_b(prevd)
        gb = [pltpu.async_copy(g_hbm.at[idxs.at[G + c]], rows[G + c], gsb)
              for c in range(G)]
        if nidxs is not None:
            fire_idx(c0n, nidxs, nidxd)
        for dsc in ga:
            dsc.wait()
        for c in range(G):
            pltpu.async_copy(rows[c], acc_sh.at[idxd.at[c]], ssa, add=True)
        for dsc in gb:
            dsc.wait()
        for c in range(G):
            pltpu.async_copy(rows[G + c], acc_sh.at[idxd.at[G + c]], ssb,
                             add=True)

    fire_idx(0, p_s, p_d)
    superstep(0, p_s, p_d, None, False, SS, q_s, q_d)

    def dbl(i, _):
        cq = pl.multiple_of(2 * SS * i + SS, 8)
        superstep(cq, q_s, q_d, p_d, True, cq + SS, p_s, p_d)
        superstep(cq + SS, p_s, p_d, q_d, True, cq + 2 * SS, q_s, q_d)
        return 0

    lax.fori_loop(0, (NSUPER - 3) // 2, dbl, 0)

    c29 = pl.multiple_of((NSUPER - 2) * SS, 8)
    superstep(c29, q_s, q_d, p_d, True, c29 + SS, p_s, p_d)
    superstep(c29 + SS, p_s, p_d, q_d, True, 0, None, None)
    drain_a(p_d)
    drain_b(p_d)

    dt1 = pltpu.async_copy(
        srcr_hbm.at[wid, pl.ds(pl.multiple_of(NSUPER * SS, 8), TAIL)],
        q_s.at[pl.ds(0, TAIL)], isem)
    dt2 = pltpu.async_copy(
        dstr_hbm.at[wid, pl.ds(pl.multiple_of(NSUPER * SS, 8), TAIL)],
        q_d.at[pl.ds(0, TAIL)], isem)
    dt1.wait()
    dt2.wait()
    gt = [pltpu.async_copy(g_hbm.at[q_s.at[c]], rows_a[c], gsa)
          for c in range(TAIL)]
    for dsc in gt:
        dsc.wait()
    st = [pltpu.async_copy(rows_a[c], acc_sh.at[q_d.at[c]], ssa, add=True)
          for c in range(TAIL)]
    for dsc in st:
        dsc.wait()

    plsc.subcore_barrier()
    pltpu.sync_copy(acc_sh.at[pl.ds(base_r, RPT)],
                    out_hbm.at[cid, pl.ds(base_r, RPT)])


R_TC = 2048


def _dinv_block(dp_ref):
    deg = jnp.sum(dp_ref[...], axis=1, keepdims=True) + 1.0
    return lax.rsqrt(deg)


def _tc1_body(x_ref, w_ref, dp_ref, o_ref):
    dinv = _dinv_block(dp_ref)
    h = jnp.dot(x_ref[...], w_ref[...], preferred_element_type=jnp.float32)
    o_ref[...] = h * dinv


def _tc2_body(acc_ref, g_ref, dp_ref, w_ref, b_ref, o_ref):
    dinv = _dinv_block(dp_ref)
    s = acc_ref[0] + acc_ref[1] + g_ref[...]
    h = jnp.maximum(s * dinv + b_ref[...], 0.0)
    o_ref[...] = jnp.dot(h, w_ref[...], preferred_element_type=jnp.float32) * dinv


def _tc3_body(acc_ref, g_ref, dp_ref, b_ref, o_ref):
    dinv = _dinv_block(dp_ref)
    s = acc_ref[0] + acc_ref[1] + g_ref[...]
    o_ref[...] = s * dinv + b_ref[...]


_row_spec = pl.BlockSpec((R_TC, D), lambda i: (i, 0))
_acc_spec = pl.BlockSpec((NC, R_TC, D), lambda i: (0, i, 0))
_dp_spec = pl.BlockSpec((R_TC, NW), lambda i: (i, 0))
_w_spec = pl.BlockSpec((D, D), lambda i: (0, 0))
_b_spec = pl.BlockSpec((1, D), lambda i: (0, 0))
_grid = (N_PAD // R_TC,)
_out_t = jax.ShapeDtypeStruct((N_PAD, D), jnp.float32)

_tc1 = pl.pallas_call(
    _tc1_body, grid=_grid,
    in_specs=[_row_spec, _w_spec, _dp_spec],
    out_specs=_row_spec, out_shape=_out_t)

_tc2 = pl.pallas_call(
    _tc2_body, grid=_grid,
    in_specs=[_acc_spec, _row_spec, _dp_spec, _w_spec, _b_spec],
    out_specs=_row_spec, out_shape=_out_t)

_tc3 = pl.pallas_call(
    _tc3_body, grid=_grid,
    in_specs=[_acc_spec, _row_spec, _dp_spec, _b_spec],
    out_specs=_row_spec, out_shape=_out_t)


def kernel(x, edge_index, W1, b1, W2, b2):
    src = edge_index[0].reshape(NW, NCHUNK, K)
    dst = edge_index[1].reshape(NW, NCHUNK, K)
    dsth = edge_index[1].reshape(NW, HROWS, 16)
    xp = jnp.pad(x, ((0, N_PAD - N), (0, 0)))
    dp = _deg_kernel(dsth).reshape(NW, N_PAD).T
    g1 = _tc1(xp, W1, dp)
    acc1 = _edge_kernel(src, dst, g1)
    g2 = _tc2(acc1, g1, dp, W2, b1.reshape(1, D))
    acc2 = _edge_kernel(src, dst, g2)
    return _tc3(acc2, g2, dp, b2.reshape(1, D))[:N]


# speedup vs baseline: 1.1699x; 1.0220x over previous
import functools

import jax
import jax.numpy as jnp
from jax import lax
from jax.experimental import pallas as pl
from jax.experimental.pallas import tpu as pltpu
from jax.experimental.pallas import tpu_sc as plsc

N = 10000
E = 320000
D = 128

NC = 2
NS = 16
EPT = E // (NC * NS)
K = 40
NCHUNK = EPT // K
N_PAD = 10240
RPT = N_PAD // NS
ZR = 160
DEG_W = 128

_mesh = plsc.VectorSubcoreMesh(core_axis_name="c", subcore_axis_name="s")


def _zero_fill(buf, rows, width):
    zv = jnp.zeros((16,), jnp.float32)

    def body(i, _):
        for j in range(width // 16):
            buf[i, pl.ds(j * 16, 16)] = zv
        return 0

    lax.fori_loop(0, rows, body, 0)


G = 4
NSUPER = 31
TAIL = NCHUNK - 2 * G * NSUPER
DEG_G = 5


NW = NC * NS
HROWS = EPT // 16
HPAD = N_PAD // 128


@functools.partial(
    pl.kernel,
    out_type=jax.ShapeDtypeStruct((NW, HPAD, 128), jnp.float32),
    mesh=_mesh,
    compiler_params=pltpu.CompilerParams(needs_layout_passes=False),
    scratch_types=[
        pltpu.VMEM((HPAD, 128), jnp.float32),
        pltpu.VMEM((HROWS, 16), jnp.int32),
        pltpu.SemaphoreType.DMA,
    ],
)
def _deg_kernel(dstr_hbm, out_hbm, hist, idxv, sem):
    cid = lax.axis_index("c")
    sid = lax.axis_index("s")
    wid = cid * NS + sid
    ld = pltpu.async_copy(dstr_hbm.at[wid], idxv, sem)

    def zrow(i, _):
        for j in range(8):
            hist[i, pl.ds(j * 16, 16)] = jnp.zeros((16,), jnp.float32)
        return 0

    lax.fori_loop(0, HPAD, zrow, 0)
    ld.wait()
    ones = jnp.ones((16,), jnp.float32)

    def row(r, _):
        n = idxv[r, :]
        hi = lax.shift_right_logical(n, 7)
        lo = lax.bitwise_and(n, 127)
        plsc.addupdate_scatter(hist, (hi, lo), ones)
        return 0

    lax.fori_loop(0, HROWS, row, 0)
    pltpu.sync_copy(hist, out_hbm.at[wid])


@functools.partial(
    pl.kernel,
    out_type=jax.ShapeDtypeStruct((NC, N_PAD, D), jnp.float32),
    mesh=_mesh,
    scratch_types=[
        pltpu.VMEM_SHARED((N_PAD, D), jnp.float32),
        pltpu.VMEM((2 * G, K), jnp.int32),
        pltpu.VMEM((2 * G, K), jnp.int32),
        pltpu.VMEM((2 * G, K), jnp.int32),
        pltpu.VMEM((2 * G, K), jnp.int32),
    ] + [pltpu.VMEM((K, D), jnp.float32) for _ in range(2 * G)] + [
        pltpu.SemaphoreType.DMA,
        pltpu.SemaphoreType.DMA,
        pltpu.SemaphoreType.DMA,
        pltpu.SemaphoreType.DMA,
        pltpu.SemaphoreType.DMA,
    ],
)
def _edge_kernel(srcr_hbm, dstr_hbm, g_hbm, out_hbm,
                 acc_sh, p_s, p_d, q_s, q_d,
                 ra0, ra1, ra2, ra3, rb0, rb1, rb2, rb3,
                 isem, gsa, gsb, ssa, ssb):
    cid = lax.axis_index("c")
    sid = lax.axis_index("s")
    wid = cid * NS + sid
    rows_a = [ra0, ra1, ra2, ra3]
    rows_b = [rb0, rb1, rb2, rb3]

    _zero_fill(ra0, K, D)
    base_r = sid * RPT
    for t in range(RPT // K):
        pltpu.sync_copy(ra0, acc_sh.at[pl.ds(base_r + t * K, K)])
    plsc.subcore_barrier()

    SS = 2 * G
    rows = rows_a + rows_b

    def fire_idx(c0, idxs, idxd):
        pltpu.async_copy(srcr_hbm.at[wid, pl.ds(c0, SS)], idxs, isem)
        pltpu.async_copy(dstr_hbm.at[wid, pl.ds(c0, SS)], idxd, isem)

    def wait_idx(c0, idxs, idxd):
        pltpu.make_async_copy(srcr_hbm.at[wid, pl.ds(c0, SS)], idxs,
                              isem).wait()
        pltpu.make_async_copy(dstr_hbm.at[wid, pl.ds(c0, SS)], idxd,
                              isem).wait()

    def drain_a(idxd):
        for c in range(G):
            pltpu.make_async_copy(rows[c], acc_sh.at[idxd.at[c]], ssa).wait()

    def drain_b(idxd):
        for c in range(G):
            pltpu.make_async_copy(rows[G + c], acc_sh.at[idxd.at[G + c]],
                                  ssb).wait()

    def superstep(c0, idxs, idxd, prevd, drains, c0n, nidxs, nidxd):
        if drains:
            drain_a(prevd)
        wait_idx(c0, idxs, idxd)
        ga = [pltpu.async_copy(g_hbm.at[idxs.at[c]], rows[c], gsa)
              for c in range(G)]
        if drains:
            drain_b(prevd)
        gb = [pltpu.async_copy(g_hbm.at[idxs.at[G + c]], rows[G + c], gsb)
              for c in range(G)]
        if nidxs is not None:
            fire_idx(c0n, nidxs, nidxd)
        for dsc in ga:
            dsc.wait()
        for c in range(G):
            pltpu.async_copy(rows[c], acc_sh.at[idxd.at[c]], ssa, add=True)
        for dsc in gb:
            dsc.wait()
        for c in range(G):
            pltpu.async_copy(rows[G + c], acc_sh.at[idxd.at[G + c]], ssb,
                             add=True)

    fire_idx(0, p_s, p_d)
    superstep(0, p_s, p_d, None, False, SS, q_s, q_d)

    def dbl(i, _):
        cq = pl.multiple_of(2 * SS * i + SS, 8)
        superstep(cq, q_s, q_d, p_d, True, cq + SS, p_s, p_d)
        superstep(cq + SS, p_s, p_d, q_d, True, cq + 2 * SS, q_s, q_d)
        return 0

    lax.fori_loop(0, (NSUPER - 3) // 2, dbl, 0)

    c29 = pl.multiple_of((NSUPER - 2) * SS, 8)
    superstep(c29, q_s, q_d, p_d, True, c29 + SS, p_s, p_d)
    superstep(c29 + SS, p_s, p_d, q_d, True, 0, None, None)
    drain_a(p_d)
    drain_b(p_d)

    dt1 = pltpu.async_copy(
        srcr_hbm.at[wid, pl.ds(pl.multiple_of(NSUPER * SS, 8), TAIL)],
        q_s.at[pl.ds(0, TAIL)], isem)
    dt2 = pltpu.async_copy(
        dstr_hbm.at[wid, pl.ds(pl.multiple_of(NSUPER * SS, 8), TAIL)],
        q_d.at[pl.ds(0, TAIL)], isem)
    dt1.wait()
    dt2.wait()
    gt = [pltpu.async_copy(g_hbm.at[q_s.at[c]], rows_a[c], gsa)
          for c in range(TAIL)]
    for dsc in gt:
        dsc.wait()
    st = [pltpu.async_copy(rows_a[c], acc_sh.at[q_d.at[c]], ssa, add=True)
          for c in range(TAIL)]
    for dsc in st:
        dsc.wait()

    plsc.subcore_barrier()
    pltpu.sync_copy(acc_sh.at[pl.ds(base_r, RPT)],
                    out_hbm.at[cid, pl.ds(base_r, RPT)])


R_TC = 2000


def _dinv_block(dp_ref):
    deg = jnp.sum(dp_ref[...], axis=1, keepdims=True) + 1.0
    return lax.rsqrt(deg)


def _tc1_body(x_ref, w_ref, dp_ref, o_ref):
    dinv = _dinv_block(dp_ref)
    h = jnp.dot(x_ref[...], w_ref[...], preferred_element_type=jnp.float32)
    o_ref[...] = h * dinv


def _tc2_body(acc_ref, g_ref, dp_ref, w_ref, b_ref, o_ref):
    dinv = _dinv_block(dp_ref)
    s = acc_ref[0] + acc_ref[1] + g_ref[...]
    h = jnp.maximum(s * dinv + b_ref[...], 0.0)
    o_ref[...] = jnp.dot(h, w_ref[...], preferred_element_type=jnp.float32) * dinv


def _tc3_body(acc_ref, g_ref, dp_ref, b_ref, o_ref):
    dinv = _dinv_block(dp_ref)
    s = acc_ref[0] + acc_ref[1] + g_ref[...]
    o_ref[...] = s * dinv + b_ref[...]


_row_spec = pl.BlockSpec((R_TC, D), lambda i: (i, 0))
_acc_spec = pl.BlockSpec((NC, R_TC, D), lambda i: (0, i, 0))
_dp_spec = pl.BlockSpec((R_TC, NW), lambda i: (i, 0))
_w_spec = pl.BlockSpec((D, D), lambda i: (0, 0))
_b_spec = pl.BlockSpec((1, D), lambda i: (0, 0))
_grid = (N // R_TC,)
_out_t = jax.ShapeDtypeStruct((N, D), jnp.float32)

_tc1 = pl.pallas_call(
    _tc1_body, grid=_grid,
    in_specs=[_row_spec, _w_spec, _dp_spec],
    out_specs=_row_spec, out_shape=_out_t)

_tc2 = pl.pallas_call(
    _tc2_body, grid=_grid,
    in_specs=[_acc_spec, _row_spec, _dp_spec, _w_spec, _b_spec],
    out_specs=_row_spec, out_shape=_out_t)

_tc3 = pl.pallas_call(
    _tc3_body, grid=_grid,
    in_specs=[_acc_spec, _row_spec, _dp_spec, _b_spec],
    out_specs=_row_spec, out_shape=_out_t)


def kernel(x, edge_index, W1, b1, W2, b2):
    src = edge_index[0].reshape(NW, NCHUNK, K)
    dst = edge_index[1].reshape(NW, NCHUNK, K)
    dsth = edge_index[1].reshape(NW, HROWS, 16)
    dp = _deg_kernel(dsth).reshape(NW, N_PAD).T
    g1 = _tc1(x, W1, dp)
    acc1 = _edge_kernel(src, dst, g1)
    g2 = _tc2(acc1, g1, dp, W2, b1.reshape(1, D))
    acc2 = _edge_kernel(src, dst, g2)
    return _tc3(acc2, g2, dp, b2.reshape(1, D))


# speedup vs baseline: 1.1782x; 1.0071x over previous
import functools

import jax
import jax.numpy as jnp
from jax import lax
from jax.experimental import pallas as pl
from jax.experimental.pallas import tpu as pltpu
from jax.experimental.pallas import tpu_sc as plsc

N = 10000
E = 320000
D = 128

NC = 2
NS = 16
EPT = E // (NC * NS)
K = 40
NCHUNK = EPT // K
N_PAD = 10240
RPT = N_PAD // NS
ZR = 160
DEG_W = 128

_mesh = plsc.VectorSubcoreMesh(core_axis_name="c", subcore_axis_name="s")


def _zero_fill(buf, rows, width):
    zv = jnp.zeros((16,), jnp.float32)

    def body(i, _):
        for j in range(width // 16):
            buf[i, pl.ds(j * 16, 16)] = zv
        return 0

    lax.fori_loop(0, rows, body, 0)


G = 4
NSUPER = 31
TAIL = NCHUNK - 2 * G * NSUPER
DEG_G = 5


NW = NC * NS
HROWS = EPT // 16
HPAD = N_PAD // 128


@functools.partial(
    pl.kernel,
    out_type=jax.ShapeDtypeStruct((NW, HPAD, 128), jnp.float32),
    mesh=_mesh,
    compiler_params=pltpu.CompilerParams(needs_layout_passes=False),
    scratch_types=[
        pltpu.VMEM((HPAD, 128), jnp.float32),
        pltpu.VMEM((HROWS, 16), jnp.int32),
        pltpu.SemaphoreType.DMA,
    ],
)
def _deg_kernel(dstr_hbm, out_hbm, hist, idxv, sem):
    cid = lax.axis_index("c")
    sid = lax.axis_index("s")
    wid = cid * NS + sid
    ld = pltpu.async_copy(dstr_hbm.at[wid], idxv, sem)

    def zrow(i, _):
        for j in range(8):
            hist[i, pl.ds(j * 16, 16)] = jnp.zeros((16,), jnp.float32)
        return 0

    lax.fori_loop(0, HPAD, zrow, 0)
    ld.wait()
    ones = jnp.ones((16,), jnp.float32)

    def row(r, _):
        n = idxv[r, :]
        hi = lax.shift_right_logical(n, 7)
        lo = lax.bitwise_and(n, 127)
        plsc.addupdate_scatter(hist, (hi, lo), ones)
        return 0

    lax.fori_loop(0, HROWS, row, 0)
    pltpu.sync_copy(hist, out_hbm.at[wid])


@functools.partial(
    pl.kernel,
    out_type=jax.ShapeDtypeStruct((NC, N_PAD, D), jnp.float32),
    mesh=_mesh,
    scratch_types=[
        pltpu.VMEM_SHARED((N_PAD, D), jnp.float32),
        pltpu.VMEM((2 * G, K), jnp.int32),
        pltpu.VMEM((2 * G, K), jnp.int32),
        pltpu.VMEM((2 * G, K), jnp.int32),
        pltpu.VMEM((2 * G, K), jnp.int32),
    ] + [pltpu.VMEM((K, D), jnp.float32) for _ in range(2 * G)] + [
        pltpu.SemaphoreType.DMA,
        pltpu.SemaphoreType.DMA,
        pltpu.SemaphoreType.DMA,
        pltpu.SemaphoreType.DMA,
        pltpu.SemaphoreType.DMA,
    ],
)
def _edge_kernel(srcr_hbm, dstr_hbm, g_hbm, out_hbm,
                 acc_sh, p_s, p_d, q_s, q_d,
                 ra0, ra1, ra2, ra3, rb0, rb1, rb2, rb3,
                 isem, gsa, gsb, ssa, ssb):
    cid = lax.axis_index("c")
    sid = lax.axis_index("s")
    wid = cid * NS + sid
    rows_a = [ra0, ra1, ra2, ra3]
    rows_b = [rb0, rb1, rb2, rb3]

    SS = 2 * G
    rows = rows_a + rows_b

    def fire_idx(c0, idxs, idxd):
        pltpu.async_copy(srcr_hbm.at[wid, pl.ds(c0, SS)], idxs, isem)
        pltpu.async_copy(dstr_hbm.at[wid, pl.ds(c0, SS)], idxd, isem)

    def wait_idx(c0, idxs, idxd):
        pltpu.make_async_copy(srcr_hbm.at[wid, pl.ds(c0, SS)], idxs,
                              isem).wait()
        pltpu.make_async_copy(dstr_hbm.at[wid, pl.ds(c0, SS)], idxd,
                              isem).wait()

    def drain_a(idxd):
        for c in range(G):
            pltpu.make_async_copy(rows[c], acc_sh.at[idxd.at[c]], ssa).wait()

    def drain_b(idxd):
        for c in range(G):
            pltpu.make_async_copy(rows[G + c], acc_sh.at[idxd.at[G + c]],
                                  ssb).wait()

    def superstep(c0, idxs, idxd, prevd, drains, c0n, nidxs, nidxd):
        if drains:
            drain_a(prevd)
        wait_idx(c0, idxs, idxd)
        ga = [pltpu.async_copy(g_hbm.at[idxs.at[c]], rows[c], gsa)
              for c in range(G)]
        if drains:
            drain_b(prevd)
        gb = [pltpu.async_copy(g_hbm.at[idxs.at[G + c]], rows[G + c], gsb)
              for c in range(G)]
        if nidxs is not None:
            fire_idx(c0n, nidxs, nidxd)
        for dsc in ga:
            dsc.wait()
        for c in range(G):
            pltpu.async_copy(rows[c], acc_sh.at[idxd.at[c]], ssa, add=True)
        for dsc in gb:
            dsc.wait()
        for c in range(G):
            pltpu.async_copy(rows[G + c], acc_sh.at[idxd.at[G + c]], ssb,
                             add=True)

    fire_idx(0, p_s, p_d)
    _zero_fill(ra0, K, D)
    base_r = sid * RPT
    zds = [pltpu.async_copy(ra0, acc_sh.at[pl.ds(base_r + t * K, K)], ssa)
           for t in range(RPT // K)]
    for dsc in zds:
        dsc.wait()
    plsc.subcore_barrier()

    superstep(0, p_s, p_d, None, False, SS, q_s, q_d)

    def dbl(i, _):
        cq = pl.multiple_of(2 * SS * i + SS, 8)
        superstep(cq, q_s, q_d, p_d, True, cq + SS, p_s, p_d)
        superstep(cq + SS, p_s, p_d, q_d, True, cq + 2 * SS, q_s, q_d)
        return 0

    lax.fori_loop(0, (NSUPER - 3) // 2, dbl, 0)

    c29 = pl.multiple_of((NSUPER - 2) * SS, 8)
    superstep(c29, q_s, q_d, p_d, True, c29 + SS, p_s, p_d)
    superstep(c29 + SS, p_s, p_d, q_d, True, 0, None, None)
    drain_a(p_d)
    drain_b(p_d)

    dt1 = pltpu.async_copy(
        srcr_hbm.at[wid, pl.ds(pl.multiple_of(NSUPER * SS, 8), TAIL)],
        q_s.at[pl.ds(0, TAIL)], isem)
    dt2 = pltpu.async_copy(
        dstr_hbm.at[wid, pl.ds(pl.multiple_of(NSUPER * SS, 8), TAIL)],
        q_d.at[pl.ds(0, TAIL)], isem)
    dt1.wait()
    dt2.wait()
    gt = [pltpu.async_copy(g_hbm.at[q_s.at[c]], rows_a[c], gsa)
          for c in range(TAIL)]
    for dsc in gt:
        dsc.wait()
    st = [pltpu.async_copy(rows_a[c], acc_sh.at[q_d.at[c]], ssa, add=True)
          for c in range(TAIL)]
    for dsc in st:
        dsc.wait()

    plsc.subcore_barrier()
    pltpu.sync_copy(acc_sh.at[pl.ds(base_r, RPT)],
                    out_hbm.at[cid, pl.ds(base_r, RPT)])


R_TC = 2000


def _dinv_block(dp_ref):
    deg = jnp.sum(dp_ref[...], axis=1, keepdims=True) + 1.0
    return lax.rsqrt(deg)


def _tc1_body(x_ref, w_ref, dp_ref, o_ref):
    dinv = _dinv_block(dp_ref)
    h = jnp.dot(x_ref[...], w_ref[...], preferred_element_type=jnp.float32)
    o_ref[...] = h * dinv


def _tc2_body(acc_ref, g_ref, dp_ref, w_ref, b_ref, o_ref):
    dinv = _dinv_block(dp_ref)
    s = acc_ref[0] + acc_ref[1] + g_ref[...]
    h = jnp.maximum(s * dinv + b_ref[...], 0.0)
    o_ref[...] = jnp.dot(h, w_ref[...], preferred_element_type=jnp.float32) * dinv


def _tc3_body(acc_ref, g_ref, dp_ref, b_ref, o_ref):
    dinv = _dinv_block(dp_ref)
    s = acc_ref[0] + acc_ref[1] + g_ref[...]
    o_ref[...] = s * dinv + b_ref[...]


_row_spec = pl.BlockSpec((R_TC, D), lambda i: (i, 0))
_acc_spec = pl.BlockSpec((NC, R_TC, D), lambda i: (0, i, 0))
_dp_spec = pl.BlockSpec((R_TC, NW), lambda i: (i, 0))
_w_spec = pl.BlockSpec((D, D), lambda i: (0, 0))
_b_spec = pl.BlockSpec((1, D), lambda i: (0, 0))
_grid = (N // R_TC,)
_out_t = jax.ShapeDtypeStruct((N, D), jnp.float32)

_tc1 = pl.pallas_call(
    _tc1_body, grid=_grid,
    in_specs=[_row_spec, _w_spec, _dp_spec],
    out_specs=_row_spec, out_shape=_out_t)

_tc2 = pl.pallas_call(
    _tc2_body, grid=_grid,
    in_specs=[_acc_spec, _row_spec, _dp_spec, _w_spec, _b_spec],
    out_specs=_row_spec, out_shape=_out_t)

_tc3 = pl.pallas_call(
    _tc3_body, grid=_grid,
    in_specs=[_acc_spec, _row_spec, _dp_spec, _b_spec],
    out_specs=_row_spec, out_shape=_out_t)


def kernel(x, edge_index, W1, b1, W2, b2):
    src = edge_index[0].reshape(NW, NCHUNK, K)
    dst = edge_index[1].reshape(NW, NCHUNK, K)
    dsth = edge_index[1].reshape(NW, HROWS, 16)
    dp = _deg_kernel(dsth).reshape(NW, N_PAD).T
    g1 = _tc1(x, W1, dp)
    acc1 = _edge_kernel(src, dst, g1)
    g2 = _tc2(acc1, g1, dp, W2, b1.reshape(1, D))
    acc2 = _edge_kernel(src, dst, g2)
    return _tc3(acc2, g2, dp, b2.reshape(1, D))


# speedup vs baseline: 1.1797x; 1.0012x over previous
import functools

import jax
import jax.numpy as jnp
from jax import lax
from jax.experimental import pallas as pl
from jax.experimental.pallas import tpu as pltpu
from jax.experimental.pallas import tpu_sc as plsc

N = 10000
E = 320000
D = 128

NC = 2
NS = 16
EPT = E // (NC * NS)
K = 40
NCHUNK = EPT // K
N_PAD = 10240
RPT = N_PAD // NS

_mesh = plsc.VectorSubcoreMesh(core_axis_name="c", subcore_axis_name="s")


def _zero_fill(buf, rows, width):
    zv = jnp.zeros((16,), jnp.float32)

    def body(i, _):
        for j in range(width // 16):
            buf[i, pl.ds(j * 16, 16)] = zv
        return 0

    lax.fori_loop(0, rows, body, 0)


G = 4
NSUPER = 31
TAIL = NCHUNK - 2 * G * NSUPER


NW = NC * NS
HROWS = EPT // 16
HPAD = N_PAD // 128


@functools.partial(
    pl.kernel,
    out_type=jax.ShapeDtypeStruct((NW, HPAD, 128), jnp.float32),
    mesh=_mesh,
    compiler_params=pltpu.CompilerParams(needs_layout_passes=False),
    scratch_types=[
        pltpu.VMEM((HPAD, 128), jnp.float32),
        pltpu.VMEM((HROWS, 16), jnp.int32),
        pltpu.SemaphoreType.DMA,
    ],
)
def _deg_kernel(dstr_hbm, out_hbm, hist, idxv, sem):
    cid = lax.axis_index("c")
    sid = lax.axis_index("s")
    wid = cid * NS + sid
    ld = pltpu.async_copy(dstr_hbm.at[wid], idxv, sem)

    def zrow(i, _):
        for j in range(8):
            hist[i, pl.ds(j * 16, 16)] = jnp.zeros((16,), jnp.float32)
        return 0

    lax.fori_loop(0, HPAD, zrow, 0)
    ld.wait()
    ones = jnp.ones((16,), jnp.float32)

    def row(r, _):
        n = idxv[r, :]
        hi = lax.shift_right_logical(n, 7)
        lo = lax.bitwise_and(n, 127)
        plsc.addupdate_scatter(hist, (hi, lo), ones)
        return 0

    lax.fori_loop(0, HROWS, row, 0)
    pltpu.sync_copy(hist, out_hbm.at[wid])


@functools.partial(
    pl.kernel,
    out_type=jax.ShapeDtypeStruct((NC, N_PAD, D), jnp.float32),
    mesh=_mesh,
    scratch_types=[
        pltpu.VMEM_SHARED((N_PAD, D), jnp.float32),
        pltpu.VMEM((2 * G, K), jnp.int32),
        pltpu.VMEM((2 * G, K), jnp.int32),
        pltpu.VMEM((2 * G, K), jnp.int32),
        pltpu.VMEM((2 * G, K), jnp.int32),
    ] + [pltpu.VMEM((K, D), jnp.float32) for _ in range(2 * G)] + [
        pltpu.SemaphoreType.DMA,
        pltpu.SemaphoreType.DMA,
        pltpu.SemaphoreType.DMA,
        pltpu.SemaphoreType.DMA,
        pltpu.SemaphoreType.DMA,
    ],
)
def _edge_kernel(srcr_hbm, dstr_hbm, g_hbm, out_hbm,
                 acc_sh, p_s, p_d, q_s, q_d,
                 ra0, ra1, ra2, ra3, rb0, rb1, rb2, rb3,
                 isem, gsa, gsb, ssa, ssb):
    cid = lax.axis_index("c")
    sid = lax.axis_index("s")
    wid = cid * NS + sid
    rows_a = [ra0, ra1, ra2, ra3]
    rows_b = [rb0, rb1, rb2, rb3]

    SS = 2 * G
    rows = rows_a + rows_b

    def fire_idx(c0, idxs, idxd):
        pltpu.async_copy(srcr_hbm.at[wid, pl.ds(c0, SS)], idxs, isem)
        pltpu.async_copy(dstr_hbm.at[wid, pl.ds(c0, SS)], idxd, isem)

    def wait_idx(c0, idxs, idxd):
        pltpu.make_async_copy(srcr_hbm.at[wid, pl.ds(c0, SS)], idxs,
                              isem).wait()
        pltpu.make_async_copy(dstr_hbm.at[wid, pl.ds(c0, SS)], idxd,
                              isem).wait()

    def drain_a(idxd):
        for c in range(G):
            pltpu.make_async_copy(rows[c], acc_sh.at[idxd.at[c]], ssa).wait()

    def drain_b(idxd):
        for c in range(G):
            pltpu.make_async_copy(rows[G + c], acc_sh.at[idxd.at[G + c]],
                                  ssb).wait()

    def superstep(c0, idxs, idxd, prevd, drains, c0n, nidxs, nidxd):
        if drains:
            drain_a(prevd)
        wait_idx(c0, idxs, idxd)
        ga = [pltpu.async_copy(g_hbm.at[idxs.at[c]], rows[c], gsa)
              for c in range(G)]
        if drains:
            drain_b(prevd)
        gb = [pltpu.async_copy(g_hbm.at[idxs.at[G + c]], rows[G + c], gsb)
              for c in range(G)]
        if nidxs is not None:
            fire_idx(c0n, nidxs, nidxd)
        for dsc in ga:
            dsc.wait()
        for c in range(G):
            pltpu.async_copy(rows[c], acc_sh.at[idxd.at[c]], ssa, add=True)
        for dsc in gb:
            dsc.wait()
        for c in range(G):
            pltpu.async_copy(rows[G + c], acc_sh.at[idxd.at[G + c]], ssb,
                             add=True)

    fire_idx(0, p_s, p_d)
    _zero_fill(ra0, K, D)
    base_r = sid * RPT
    zds = [pltpu.async_copy(ra0, acc_sh.at[pl.ds(base_r + t * K, K)], ssa)
           for t in range(RPT // K)]
    for dsc in zds:
        dsc.wait()
    plsc.subcore_barrier()

    superstep(0, p_s, p_d, None, False, SS, q_s, q_d)

    def dbl(i, _):
        cq = pl.multiple_of(2 * SS * i + SS, 8)
        superstep(cq, q_s, q_d, p_d, True, cq + SS, p_s, p_d)
        superstep(cq + SS, p_s, p_d, q_d, True, cq + 2 * SS, q_s, q_d)
        return 0

    lax.fori_loop(0, (NSUPER - 3) // 2, dbl, 0)

    c29 = pl.multiple_of((NSUPER - 2) * SS, 8)
    superstep(c29, q_s, q_d, p_d, True, c29 + SS, p_s, p_d)
    superstep(c29 + SS, p_s, p_d, q_d, True, 0, None, None)
    drain_a(p_d)
    drain_b(p_d)

    dt1 = pltpu.async_copy(
        srcr_hbm.at[wid, pl.ds(pl.multiple_of(NSUPER * SS, 8), TAIL)],
        q_s.at[pl.ds(0, TAIL)], isem)
    dt2 = pltpu.async_copy(
        dstr_hbm.at[wid, pl.ds(pl.multiple_of(NSUPER * SS, 8), TAIL)],
        q_d.at[pl.ds(0, TAIL)], isem)
    dt1.wait()
    dt2.wait()
    gt = [pltpu.async_copy(g_hbm.at[q_s.at[c]], rows_a[c], gsa)
          for c in range(TAIL)]
    for dsc in gt:
        dsc.wait()
    st = [pltpu.async_copy(rows_a[c], acc_sh.at[q_d.at[c]], ssa, add=True)
          for c in range(TAIL)]
    for dsc in st:
        dsc.wait()

    plsc.subcore_barrier()
    pltpu.sync_copy(acc_sh.at[pl.ds(base_r, RPT)],
                    out_hbm.at[cid, pl.ds(base_r, RPT)])


R_TC = 2000


def _dinv_block(dp_ref):
    deg = jnp.sum(dp_ref[...], axis=1, keepdims=True) + 1.0
    return lax.rsqrt(deg)


def _tc1_body(x_ref, w_ref, dp_ref, o_ref):
    dinv = _dinv_block(dp_ref)
    h = jnp.dot(x_ref[...], w_ref[...], preferred_element_type=jnp.float32)
    o_ref[...] = h * dinv


def _tc2_body(acc_ref, g_ref, dp_ref, w_ref, b_ref, o_ref):
    dinv = _dinv_block(dp_ref)
    s = acc_ref[0] + acc_ref[1] + g_ref[...]
    h = jnp.maximum(s * dinv + b_ref[...], 0.0)
    o_ref[...] = jnp.dot(h, w_ref[...], preferred_element_type=jnp.float32) * dinv


def _tc3_body(acc_ref, g_ref, dp_ref, b_ref, o_ref):
    dinv = _dinv_block(dp_ref)
    s = acc_ref[0] + acc_ref[1] + g_ref[...]
    o_ref[...] = s * dinv + b_ref[...]


_row_spec = pl.BlockSpec((R_TC, D), lambda i: (i, 0))
_acc_spec = pl.BlockSpec((NC, R_TC, D), lambda i: (0, i, 0))
_dp_spec = pl.BlockSpec((R_TC, NW), lambda i: (i, 0))
_w_spec = pl.BlockSpec((D, D), lambda i: (0, 0))
_b_spec = pl.BlockSpec((1, D), lambda i: (0, 0))
_grid = (N // R_TC,)
_out_t = jax.ShapeDtypeStruct((N, D), jnp.float32)

_tc1 = pl.pallas_call(
    _tc1_body, grid=_grid,
    in_specs=[_row_spec, _w_spec, _dp_spec],
    out_specs=_row_spec, out_shape=_out_t)

_tc2 = pl.pallas_call(
    _tc2_body, grid=_grid,
    in_specs=[_acc_spec, _row_spec, _dp_spec, _w_spec, _b_spec],
    out_specs=_row_spec, out_shape=_out_t)

_tc3 = pl.pallas_call(
    _tc3_body, grid=_grid,
    in_specs=[_acc_spec, _row_spec, _dp_spec, _b_spec],
    out_specs=_row_spec, out_shape=_out_t)


def kernel(x, edge_index, W1, b1, W2, b2):
    src = edge_index[0].reshape(NW, NCHUNK, K)
    dst = edge_index[1].reshape(NW, NCHUNK, K)
    dsth = edge_index[1].reshape(NW, HROWS, 16)
    dp = _deg_kernel(dsth).reshape(NW, N_PAD).T
    g1 = _tc1(x, W1, dp)
    acc1 = _edge_kernel(src, dst, g1)
    g2 = _tc2(acc1, g1, dp, W2, b1.reshape(1, D))
    acc2 = _edge_kernel(src, dst, g2)
    return _tc3(acc2, g2, dp, b2.reshape(1, D))
